# baseline (speedup 1.0000x reference)
import functools

import jax
import jax.numpy as jnp
from jax import lax
from jax.experimental import pallas as pl
from jax.experimental.pallas import tpu as pltpu
from jax.experimental.pallas import tpu_sc as plsc

N = 10000
D = 128
E = 320000
CHUNK = 128
NSUB = 16
NT = 160
EC = NSUB * NT
ECP = NSUB * (NT + 2)
ROWS_PER_SUB = 624


def _dense_body(mean_ref, var_ref, wmT_ref, bm_ref, wvT_ref, bv_ref,
                m_ref, v_ref):
    m = jnp.dot(mean_ref[...], wmT_ref[...],
                preferred_element_type=jnp.float32) + bm_ref[...]
    v = jnp.dot(var_ref[...], wvT_ref[...],
                preferred_element_type=jnp.float32) + bv_ref[...]
    m = jnp.where(m > 0.0, m, jnp.exp(jnp.minimum(m, 0.0)) - 1.0)
    v = jnp.maximum(v, 0.0)
    att = jnp.exp(-v)
    m_ref[...] = m * att
    v_ref[...] = v * att * att


def _dense(mean, var, wmT, bm, wvT, bv):
    return pl.pallas_call(
        _dense_body,
        out_shape=(jax.ShapeDtypeStruct((N, D), jnp.float32),
                   jax.ShapeDtypeStruct((N, D), jnp.float32)),
    )(mean, var, wmT, bm, wvT, bv)


def _pack_edges(col, row, val):
    pad = ECP * CHUNK - E
    colp = jnp.pad(col, (0, pad))
    rowp = jnp.pad(row, (0, pad))
    vali = jax.lax.bitcast_convert_type(val, jnp.int32)
    valp = jnp.pad(vali, (0, pad))
    return jnp.stack([colp.reshape(-1, CHUNK), rowp.reshape(-1, CHUNK),
                      valp.reshape(-1, CHUNK)], axis=1)


def _sc_spmm(m_s, v_s, ed_m, ed_v):
    mesh = plsc.VectorSubcoreMesh(core_axis_name="c", subcore_axis_name="s")

    @functools.partial(
        pl.kernel,
        out_type=(jax.ShapeDtypeStruct((N, D), jnp.float32),
                  jax.ShapeDtypeStruct((N, D), jnp.float32)),
        mesh=mesh,
        compiler_params=pltpu.CompilerParams(needs_layout_passes=False),
        scratch_types=[
            pltpu.VMEM_SHARED((N, D), jnp.float32),
            pltpu.VMEM((3, CHUNK), jnp.int32),
            pltpu.VMEM((3, CHUNK), jnp.int32),
            pltpu.VMEM((CHUNK, D), jnp.float32),
            pltpu.VMEM((CHUNK, D), jnp.float32),
            pltpu.VMEM((CHUNK,), jnp.int32),
            pltpu.VMEM((CHUNK,), jnp.int32),
            pltpu.VMEM((CHUNK, D), jnp.float32),
            pltpu.SemaphoreType.DMA,
            pltpu.SemaphoreType.DMA,
            pltpu.SemaphoreType.DMA,
            pltpu.SemaphoreType.DMA,
            pltpu.SemaphoreType.DMA,
            pltpu.SemaphoreType.DMA,
        ],
    )
    def sc_kernel(ms_hbm, vs_hbm, edm_hbm, edv_hbm, mo_hbm, vo_hbm,
                  acc, eb0, eb1, rb0, rb1, sb0, sb1, zerov,
                  se0, se1, sg0, sg1, ss0, ss1):
        cid = lax.axis_index("c")
        sid = lax.axis_index("s")
        base = sid * ROWS_PER_SUB
        eb = (eb0, eb1)
        rb = (rb0, rb1)
        sb = (sb0, sb1)
        se = (se0, se1)
        sg = (sg0, sg1)
        ss = (ss0, ss1)

        z16 = jnp.zeros((16,), jnp.float32)

        @pl.loop(0, CHUNK)
        def _(i):
            for j in range(8):
                zerov[i, pl.ds(j * 16, 16)] = z16

        @pl.loop(0, 4)
        def _(t):
            pltpu.sync_copy(zerov, acc.at[pl.ds(base + t * CHUNK, CHUNK)])
        pltpu.sync_copy(zerov.at[pl.ds(0, ROWS_PER_SUB - 4 * CHUNK)],
                        acc.at[pl.ds(base + 4 * CHUNK, ROWS_PER_SUB - 4 * CHUNK)])

        @pl.when(sid == NSUB - 1)
        def _():
            pltpu.sync_copy(zerov.at[pl.ds(0, 16)],
                            acc.at[pl.ds(NSUB * ROWS_PER_SUB, 16)])
        plsc.subcore_barrier()

        def process(x_hbm, ed_hbm, out_hbm):
            def chunk_of(t):
                return sid + t * NSUB

            def fire_e(t, b):
                pltpu.async_copy(ed_hbm.at[chunk_of(t)], eb[b], se[b])

            def wait_e(b):
                pltpu.make_async_copy(ed_hbm.at[0], eb[b], se[b]).wait()

            def fire_g(b):
                pltpu.async_copy(x_hbm.at[eb[b].at[0]], rb[b], sg[b])

            def wait_g(b):
                pltpu.make_async_copy(x_hbm.at[eb[b].at[0]], rb[b],
                                      sg[b]).wait()

            def fire_s(b):
                pltpu.async_copy(rb[b], acc.at[sb[b]], ss[b], add=True)

            def wait_s(b):
                pltpu.make_async_copy(rb[b], acc.at[sb[b]], ss[b]).wait()

            def scale(b):
                rbx, ebx, sbx = rb[b], eb[b], sb[b]

                @pl.loop(0, 8)
                def _(g):
                    off = g * 16
                    vals = plsc.bitcast(ebx[2, pl.ds(off, 16)], jnp.float32)
                    sbx[pl.ds(off, 16)] = ebx[1, pl.ds(off, 16)]
                    for k in range(16):
                        vk = jnp.full((16,), vals[k])
                        e = off + k
                        for j in range(8):
                            slc = (e, pl.ds(j * 16, 16))
                            rbx[slc] = rbx[slc] * vk

            @pl.loop(0, NT)
            def _(t):
                fire_e(t, 0)
                wait_e(0)
                pltpu.async_copy(x_hbm.at[eb[0].at[0]], rb[0], sg[0]).wait()
                scale(0)
                pltpu.sync_copy(rb[0], acc.at[sb[0]], add=True)

            plsc.subcore_barrier()

            @pl.loop(0, 4)
            def _(t):
                pltpu.sync_copy(acc.at[pl.ds(base + t * CHUNK, CHUNK)],
                                out_hbm.at[pl.ds(base + t * CHUNK, CHUNK)])
            pltpu.sync_copy(acc.at[pl.ds(base + 4 * CHUNK, ROWS_PER_SUB - 4 * CHUNK)],
                            out_hbm.at[pl.ds(base + 4 * CHUNK, ROWS_PER_SUB - 4 * CHUNK)])

            @pl.when(sid == NSUB - 1)
            def _():
                pltpu.sync_copy(acc.at[pl.ds(NSUB * ROWS_PER_SUB, 16)],
                                out_hbm.at[pl.ds(NSUB * ROWS_PER_SUB, 16)])

        @pl.when(cid == 0)
        def _():
            process(ms_hbm, edm_hbm, mo_hbm)

        @pl.when(cid == 1)
        def _():
            process(vs_hbm, edv_hbm, vo_hbm)

    return sc_kernel(m_s, v_s, ed_m, ed_v)


def kernel(mean, var, edge_index, adj0_values, adj1_values,
           W_mean, b_mean, W_var, b_var):
    m_s, v_s = _dense(mean, var, W_mean.T, b_mean[None, :], W_var.T,
                      b_var[None, :])
    row = edge_index[0]
    col = edge_index[1]
    ed_m = _pack_edges(col, row, adj0_values)
    ed_v = _pack_edges(col, row, adj1_values)
    return _sc_spmm(m_s, v_s, ed_m, ed_v)


# speedup vs baseline: 1.1413x; 1.1413x over previous
import functools

import jax
import jax.numpy as jnp
from jax import lax
from jax.experimental import pallas as pl
from jax.experimental.pallas import tpu as pltpu
from jax.experimental.pallas import tpu_sc as plsc

N = 10000
D = 128
E = 320000
CHUNK = 128
NSUB = 16
NT = 160
EC = NSUB * NT
ECP = NSUB * (NT + 2)
ROWS_PER_SUB = 624


def _dense_body(mean_ref, var_ref, wmT_ref, bm_ref, wvT_ref, bv_ref,
                m_ref, v_ref):
    m = jnp.dot(mean_ref[...], wmT_ref[...],
                preferred_element_type=jnp.float32) + bm_ref[...]
    v = jnp.dot(var_ref[...], wvT_ref[...],
                preferred_element_type=jnp.float32) + bv_ref[...]
    m = jnp.where(m > 0.0, m, jnp.exp(jnp.minimum(m, 0.0)) - 1.0)
    v = jnp.maximum(v, 0.0)
    att = jnp.exp(-v)
    m_ref[...] = m * att
    v_ref[...] = v * att * att


def _dense(mean, var, wmT, bm, wvT, bv):
    return pl.pallas_call(
        _dense_body,
        out_shape=(jax.ShapeDtypeStruct((N, D), jnp.float32),
                   jax.ShapeDtypeStruct((N, D), jnp.float32)),
    )(mean, var, wmT, bm, wvT, bv)


def _pack_edges(col, row, val):
    pad = ECP * CHUNK - E
    colp = jnp.pad(col, (0, pad)).reshape(-1, CHUNK)
    rowp = jnp.pad(row, (0, pad)).reshape(-1, CHUNK)
    vali = jax.lax.bitcast_convert_type(val, jnp.int32)
    valp = jnp.pad(vali, (0, pad)).reshape(-1, CHUNK)
    z = jnp.zeros_like(colp)
    return jnp.stack([colp, rowp, valp, z, z, z, z, z],
                     axis=1).reshape(-1, CHUNK)


def _sc_spmm(m_s, v_s, ed_m, ed_v):
    mesh = plsc.VectorSubcoreMesh(core_axis_name="c", subcore_axis_name="s")

    @functools.partial(
        pl.kernel,
        out_type=(jax.ShapeDtypeStruct((N, D), jnp.float32),
                  jax.ShapeDtypeStruct((N, D), jnp.float32)),
        mesh=mesh,
        compiler_params=pltpu.CompilerParams(needs_layout_passes=False),
        scratch_types=[
            pltpu.VMEM_SHARED((N, D), jnp.float32),
            pltpu.VMEM((8, CHUNK), jnp.int32),
            pltpu.VMEM((8, CHUNK), jnp.int32),
            pltpu.VMEM((CHUNK, D), jnp.float32),
            pltpu.VMEM((CHUNK, D), jnp.float32),
            pltpu.VMEM((CHUNK,), jnp.int32),
            pltpu.VMEM((CHUNK,), jnp.int32),
            pltpu.SemaphoreType.DMA,
            pltpu.SemaphoreType.DMA,
            pltpu.SemaphoreType.DMA,
            pltpu.SemaphoreType.DMA,
            pltpu.SemaphoreType.DMA,
            pltpu.SemaphoreType.DMA,
        ],
    )
    def sc_kernel(ms_hbm, vs_hbm, edm_hbm, edv_hbm, mo_hbm, vo_hbm,
                  acc, eb0, eb1, rb0, rb1, sb0, sb1,
                  se0, se1, sg0, sg1, ss0, ss1):
        cid = lax.axis_index("c")
        sid = lax.axis_index("s")
        base = sid * ROWS_PER_SUB
        eb = (eb0, eb1)
        rb = (rb0, rb1)
        sb = (sb0, sb1)
        se = (se0, se1)
        sg = (sg0, sg1)
        ss = (ss0, ss1)

        z16 = jnp.zeros((16,), jnp.float32)

        @pl.loop(0, CHUNK)
        def _(i):
            for j in range(8):
                rb0[i, pl.ds(j * 16, 16)] = z16

        @pl.loop(0, 4)
        def _(t):
            pltpu.sync_copy(rb0, acc.at[pl.ds(base + t * CHUNK, CHUNK)])
        pltpu.sync_copy(rb0.at[pl.ds(0, ROWS_PER_SUB - 4 * CHUNK)],
                        acc.at[pl.ds(base + 4 * CHUNK, ROWS_PER_SUB - 4 * CHUNK)])

        @pl.when(sid == NSUB - 1)
        def _():
            pltpu.sync_copy(rb0.at[pl.ds(0, 16)],
                            acc.at[pl.ds(NSUB * ROWS_PER_SUB, 16)])
        plsc.subcore_barrier()

        def process(x_hbm, ed_hbm, out_hbm):
            def chunk_of(t):
                return sid + t * NSUB

            def fire_e(t, b):
                pltpu.async_copy(ed_hbm.at[pl.ds(8 * chunk_of(t), 8)],
                                 eb[b], se[b])

            def wait_e(b):
                pltpu.make_async_copy(ed_hbm.at[pl.ds(0, 8)], eb[b],
                                      se[b]).wait()

            def fire_g(b):
                return pltpu.async_copy(x_hbm.at[eb[b].at[0]], rb[b], sg[b])

            def fire_s(b):
                return pltpu.async_copy(rb[b], acc.at[sb[b]], ss[b],
                                        add=True)

            def scale(b):
                rbx, ebx, sbx = rb[b], eb[b], sb[b]

                @pl.loop(0, 8)
                def _(g):
                    off = g * 16
                    vals = plsc.bitcast(ebx[2, pl.ds(off, 16)], jnp.float32)
                    sbx[pl.ds(off, 16)] = ebx[1, pl.ds(off, 16)]
                    for k in range(16):
                        vk = jnp.full((16,), vals[k])
                        e = off + k
                        for j in range(8):
                            slc = (e, pl.ds(j * 16, 16))
                            rbx[slc] = rbx[slc] * vk

            fire_e(0, 0)
            fire_e(1, 1)

            @pl.loop(0, NT // 2)
            def _(u):
                t = 2 * u
                wait_e(0)
                d_g0 = fire_g(0)
                wait_e(1)
                d_g1 = fire_g(1)
                d_g0.wait()
                scale(0)
                d_s0 = fire_s(0)
                fire_e(t + 2, 0)
                d_g1.wait()
                scale(1)
                d_s1 = fire_s(1)
                fire_e(t + 3, 1)
                d_s0.wait()
                d_s1.wait()

            wait_e(0)
            wait_e(1)

            plsc.subcore_barrier()

            @pl.loop(0, 4)
            def _(t):
                pltpu.sync_copy(acc.at[pl.ds(base + t * CHUNK, CHUNK)],
                                out_hbm.at[pl.ds(base + t * CHUNK, CHUNK)])
            pltpu.sync_copy(acc.at[pl.ds(base + 4 * CHUNK, ROWS_PER_SUB - 4 * CHUNK)],
                            out_hbm.at[pl.ds(base + 4 * CHUNK, ROWS_PER_SUB - 4 * CHUNK)])

            @pl.when(sid == NSUB - 1)
            def _():
                pltpu.sync_copy(acc.at[pl.ds(NSUB * ROWS_PER_SUB, 16)],
                                out_hbm.at[pl.ds(NSUB * ROWS_PER_SUB, 16)])

        @pl.when(cid == 0)
        def _():
            process(ms_hbm, edm_hbm, mo_hbm)

        @pl.when(cid == 1)
        def _():
            process(vs_hbm, edv_hbm, vo_hbm)

    return sc_kernel(m_s, v_s, ed_m, ed_v)


def kernel(mean, var, edge_index, adj0_values, adj1_values,
           W_mean, b_mean, W_var, b_var):
    m_s, v_s = _dense(mean, var, W_mean.T, b_mean[None, :], W_var.T,
                      b_var[None, :])
    row = edge_index[0]
    col = edge_index[1]
    ed_m = _pack_edges(col, row, adj0_values)
    ed_v = _pack_edges(col, row, adj1_values)
    return _sc_spmm(m_s, v_s, ed_m, ed_v)


# speedup vs baseline: 1.2288x; 1.0767x over previous
import functools

import jax
import jax.numpy as jnp
from jax import lax
from jax.experimental import pallas as pl
from jax.experimental.pallas import tpu as pltpu
from jax.experimental.pallas import tpu_sc as plsc

N = 10000
D = 128
E = 320000
CHUNK = 128
NSUB = 16
NT = 160
K = 8
EC = NSUB * NT
ECP = NSUB * (NT + 4)
ROWS_PER_SUB = 624


def _dense_body(mean_ref, var_ref, wmT_ref, bm_ref, wvT_ref, bv_ref,
                m_ref, v_ref):
    m = jnp.dot(mean_ref[...], wmT_ref[...],
                preferred_element_type=jnp.float32) + bm_ref[...]
    v = jnp.dot(var_ref[...], wvT_ref[...],
                preferred_element_type=jnp.float32) + bv_ref[...]
    m = jnp.where(m > 0.0, m, jnp.exp(jnp.minimum(m, 0.0)) - 1.0)
    v = jnp.maximum(v, 0.0)
    att = jnp.exp(-v)
    m_ref[...] = m * att
    v_ref[...] = v * att * att


def _dense(mean, var, wmT, bm, wvT, bv):
    return pl.pallas_call(
        _dense_body,
        out_shape=(jax.ShapeDtypeStruct((N, D), jnp.float32),
                   jax.ShapeDtypeStruct((N, D), jnp.float32)),
    )(mean, var, wmT, bm, wvT, bv)


def _pack_edges(col, row, val):
    pad = ECP * CHUNK - E
    colp = jnp.pad(col, (0, pad)).reshape(-1, CHUNK)
    rowp = jnp.pad(row, (0, pad)).reshape(-1, CHUNK)
    vali = jax.lax.bitcast_convert_type(val, jnp.int32)
    valp = jnp.pad(vali, (0, pad)).reshape(-1, CHUNK)
    z = jnp.zeros_like(colp)
    return jnp.stack([colp, rowp, valp, z, z, z, z, z],
                     axis=1).reshape(-1, CHUNK)


def _sc_spmm(m_s, v_s, ed_m, ed_v):
    mesh = plsc.VectorSubcoreMesh(core_axis_name="c", subcore_axis_name="s")

    @functools.partial(
        pl.kernel,
        out_type=(jax.ShapeDtypeStruct((N, D), jnp.float32),
                  jax.ShapeDtypeStruct((N, D), jnp.float32)),
        mesh=mesh,
        compiler_params=pltpu.CompilerParams(needs_layout_passes=False),
        scratch_types=[
            pltpu.VMEM_SHARED((N, D), jnp.float32),
            pltpu.VMEM((8, CHUNK), jnp.int32),
            pltpu.VMEM((8, CHUNK), jnp.int32),
            pltpu.VMEM((8, CHUNK), jnp.int32),
            pltpu.VMEM((8, CHUNK), jnp.int32),
            pltpu.VMEM((CHUNK, D), jnp.float32),
            pltpu.VMEM((CHUNK, D), jnp.float32),
            pltpu.VMEM((CHUNK,), jnp.int32),
            pltpu.VMEM((CHUNK,), jnp.int32),
            pltpu.SemaphoreType.DMA,
            pltpu.SemaphoreType.DMA,
            pltpu.SemaphoreType.DMA,
            pltpu.SemaphoreType.DMA,
            pltpu.SemaphoreType.DMA,
            pltpu.SemaphoreType.DMA,
            pltpu.SemaphoreType.DMA,
            pltpu.SemaphoreType.DMA,
        ],
    )
    def sc_kernel(ms_hbm, vs_hbm, edm_hbm, edv_hbm, mo_hbm, vo_hbm,
                  acc, eb0, eb1, eb2, eb3, rb0, rb1, sb0, sb1,
                  se0, se1, se2, se3, sg0, sg1, ss0, ss1):
        cid = lax.axis_index("c")
        sid = lax.axis_index("s")
        base = sid * ROWS_PER_SUB
        eb = (eb0, eb1, eb2, eb3)
        rb = (rb0, rb1)
        sb = (sb0, sb1)
        se = (se0, se1, se2, se3)
        sg = (sg0, sg1)
        ss = (ss0, ss1)

        z16 = jnp.zeros((16,), jnp.float32)

        @pl.loop(0, CHUNK)
        def _(i):
            for j in range(8):
                rb0[i, pl.ds(j * 16, 16)] = z16

        @pl.loop(0, 4)
        def _(t):
            pltpu.sync_copy(rb0, acc.at[pl.ds(base + t * CHUNK, CHUNK)])
        pltpu.sync_copy(rb0.at[pl.ds(0, ROWS_PER_SUB - 4 * CHUNK)],
                        acc.at[pl.ds(base + 4 * CHUNK, ROWS_PER_SUB - 4 * CHUNK)])

        @pl.when(sid == NSUB - 1)
        def _():
            pltpu.sync_copy(rb0.at[pl.ds(0, 16)],
                            acc.at[pl.ds(NSUB * ROWS_PER_SUB, 16)])
        plsc.subcore_barrier()

        def process(x_hbm, ed_hbm, out_hbm):
            def chunk_of(t):
                return sid + t * NSUB

            def fire_e(t, b):
                pltpu.async_copy(ed_hbm.at[pl.ds(8 * chunk_of(t), 8)],
                                 eb[b], se[b])

            def wait_e(b):
                pltpu.make_async_copy(ed_hbm.at[pl.ds(0, 8)], eb[b],
                                      se[b]).wait()

            def fire_g(b4, b2):
                return pltpu.async_copy(x_hbm.at[eb[b4].at[0]], rb[b2],
                                        sg[b2])

            def fire_s(b2):
                return pltpu.async_copy(rb[b2], acc.at[sb[b2]], ss[b2],
                                        add=True)

            def scale(b4, b2):
                rbx, ebx, sbx = rb[b2], eb[b4], sb[b2]

                @pl.loop(0, 8)
                def _(g):
                    off = g * 16
                    vals = plsc.bitcast(ebx[2, pl.ds(off, 16)], jnp.float32)
                    sbx[pl.ds(off, 16)] = ebx[1, pl.ds(off, 16)]
                    for k in range(16):
                        vk = jnp.full((16,), vals[k])
                        e = off + k
                        for j in range(8):
                            slc = (e, pl.ds(j * 16, 16))
                            rbx[slc] = rbx[slc] * vk

            for i in range(4):
                fire_e(i, i)

            @pl.loop(0, NT // K)
            def _(u):
                t0 = u * K
                dg = [None] * K
                ds = [None] * K
                for i in range(K):
                    if i >= 2:
                        ds[i - 2].wait()
                    wait_e(i % 4)
                    dg[i] = fire_g(i % 4, i % 2)
                    if i >= 1:
                        dg[i - 1].wait()
                        scale((i - 1) % 4, (i - 1) % 2)
                        ds[i - 1] = fire_s((i - 1) % 2)
                        fire_e(t0 + i + 3, (i - 1) % 4)
                dg[K - 1].wait()
                scale((K - 1) % 4, (K - 1) % 2)
                ds[K - 1] = fire_s((K - 1) % 2)
                fire_e(t0 + K + 3, (K - 1) % 4)
                ds[K - 2].wait()
                ds[K - 1].wait()

            for b in range(4):
                wait_e(b)

            plsc.subcore_barrier()

            @pl.loop(0, 4)
            def _(t):
                pltpu.sync_copy(acc.at[pl.ds(base + t * CHUNK, CHUNK)],
                                out_hbm.at[pl.ds(base + t * CHUNK, CHUNK)])
            pltpu.sync_copy(acc.at[pl.ds(base + 4 * CHUNK, ROWS_PER_SUB - 4 * CHUNK)],
                            out_hbm.at[pl.ds(base + 4 * CHUNK, ROWS_PER_SUB - 4 * CHUNK)])

            @pl.when(sid == NSUB - 1)
            def _():
                pltpu.sync_copy(acc.at[pl.ds(NSUB * ROWS_PER_SUB, 16)],
                                out_hbm.at[pl.ds(NSUB * ROWS_PER_SUB, 16)])

        @pl.when(cid == 0)
        def _():
            process(ms_hbm, edm_hbm, mo_hbm)

        @pl.when(cid == 1)
        def _():
            process(vs_hbm, edv_hbm, vo_hbm)

    return sc_kernel(m_s, v_s, ed_m, ed_v)


def kernel(mean, var, edge_index, adj0_values, adj1_values,
           W_mean, b_mean, W_var, b_var):
    m_s, v_s = _dense(mean, var, W_mean.T, b_mean[None, :], W_var.T,
                      b_var[None, :])
    row = edge_index[0]
    col = edge_index[1]
    ed_m = _pack_edges(col, row, adj0_values)
    ed_v = _pack_edges(col, row, adj1_values)
    return _sc_spmm(m_s, v_s, ed_m, ed_v)


# speedup vs baseline: 1.3392x; 1.0898x over previous
import functools

import jax
import jax.numpy as jnp
from jax import lax
from jax.experimental import pallas as pl
from jax.experimental.pallas import tpu as pltpu
from jax.experimental.pallas import tpu_sc as plsc

N = 10000
D = 128
E = 320000
CHUNK = 128
NSUB = 16
NT = 160
K = 8
EC = NSUB * NT
ECP = NSUB * (NT + 4)
ROWS_PER_SUB = 624


def _dense_body(mean_ref, var_ref, wmT_ref, bm_ref, wvT_ref, bv_ref,
                m_ref, v_ref):
    m = jnp.dot(mean_ref[...], wmT_ref[...],
                preferred_element_type=jnp.float32) + bm_ref[...]
    v = jnp.dot(var_ref[...], wvT_ref[...],
                preferred_element_type=jnp.float32) + bv_ref[...]
    m = jnp.where(m > 0.0, m, jnp.exp(jnp.minimum(m, 0.0)) - 1.0)
    v = jnp.maximum(v, 0.0)
    att = jnp.exp(-v)
    m_ref[...] = m * att
    v_ref[...] = v * att * att


def _dense(mean, var, wmT, bm, wvT, bv):
    return pl.pallas_call(
        _dense_body,
        out_shape=(jax.ShapeDtypeStruct((N, D), jnp.float32),
                   jax.ShapeDtypeStruct((N, D), jnp.float32)),
    )(mean, var, wmT, bm, wvT, bv)


def _pack_edges(col, row, val):
    pad = ECP * CHUNK - E
    colp = jnp.pad(col, (0, pad)).reshape(-1, CHUNK)
    rowp = jnp.pad(row, (0, pad)).reshape(-1, CHUNK)
    vali = jax.lax.bitcast_convert_type(val, jnp.int32)
    valp = jnp.pad(vali, (0, pad)).reshape(-1, CHUNK)
    return jnp.stack([colp, rowp, valp], axis=1)


def _sc_spmm(m_s, v_s, ed_m, ed_v):
    mesh = plsc.VectorSubcoreMesh(core_axis_name="c", subcore_axis_name="s")

    @functools.partial(
        pl.kernel,
        out_type=(jax.ShapeDtypeStruct((N, D), jnp.float32),
                  jax.ShapeDtypeStruct((N, D), jnp.float32)),
        mesh=mesh,
        compiler_params=pltpu.CompilerParams(needs_layout_passes=False),
        scratch_types=[
            pltpu.VMEM_SHARED((N, D), jnp.float32),
            pltpu.VMEM((3, CHUNK), jnp.int32),
            pltpu.VMEM((3, CHUNK), jnp.int32),
            pltpu.VMEM((3, CHUNK), jnp.int32),
            pltpu.VMEM((3, CHUNK), jnp.int32),
            pltpu.VMEM((CHUNK, D), jnp.float32),
            pltpu.VMEM((CHUNK, D), jnp.float32),
            pltpu.VMEM((CHUNK,), jnp.int32),
            pltpu.VMEM((CHUNK,), jnp.int32),
            pltpu.SemaphoreType.DMA,
            pltpu.SemaphoreType.DMA,
            pltpu.SemaphoreType.DMA,
            pltpu.SemaphoreType.DMA,
            pltpu.SemaphoreType.DMA,
            pltpu.SemaphoreType.DMA,
            pltpu.SemaphoreType.DMA,
            pltpu.SemaphoreType.DMA,
        ],
    )
    def sc_kernel(ms_hbm, vs_hbm, edm_hbm, edv_hbm, mo_hbm, vo_hbm,
                  acc, eb0, eb1, eb2, eb3, rb0, rb1, sb0, sb1,
                  se0, se1, se2, se3, sg0, sg1, ss0, ss1):
        cid = lax.axis_index("c")
        sid = lax.axis_index("s")
        base = sid * ROWS_PER_SUB
        eb = (eb0, eb1, eb2, eb3)
        rb = (rb0, rb1)
        sb = (sb0, sb1)
        se = (se0, se1, se2, se3)
        sg = (sg0, sg1)
        ss = (ss0, ss1)

        z16 = jnp.zeros((16,), jnp.float32)

        @pl.loop(0, CHUNK)
        def _(i):
            for j in range(8):
                rb0[i, pl.ds(j * 16, 16)] = z16

        @pl.loop(0, 4)
        def _(t):
            pltpu.sync_copy(rb0, acc.at[pl.ds(base + t * CHUNK, CHUNK)])
        pltpu.sync_copy(rb0.at[pl.ds(0, ROWS_PER_SUB - 4 * CHUNK)],
                        acc.at[pl.ds(base + 4 * CHUNK, ROWS_PER_SUB - 4 * CHUNK)])

        @pl.when(sid == NSUB - 1)
        def _():
            pltpu.sync_copy(rb0.at[pl.ds(0, 16)],
                            acc.at[pl.ds(NSUB * ROWS_PER_SUB, 16)])
        plsc.subcore_barrier()

        def process(x_hbm, ed_hbm, out_hbm):
            def chunk_of(t):
                return sid + t * NSUB

            def fire_e(t, b):
                pltpu.async_copy(ed_hbm.at[chunk_of(t)], eb[b], se[b])

            def wait_e(b):
                pltpu.make_async_copy(ed_hbm.at[0], eb[b], se[b]).wait()

            def fire_g(b4, b2):
                return pltpu.async_copy(x_hbm.at[eb[b4].at[0]], rb[b2],
                                        sg[b2])

            def fire_s(b2):
                return pltpu.async_copy(rb[b2], acc.at[sb[b2]], ss[b2],
                                        add=True)

            def scale(b4, b2):
                rbx, ebx, sbx = rb[b2], eb[b4], sb[b2]

                @pl.loop(0, 8)
                def _(g):
                    off = g * 16
                    vals = plsc.bitcast(ebx[2, pl.ds(off, 16)], jnp.float32)
                    sbx[pl.ds(off, 16)] = ebx[1, pl.ds(off, 16)]
                    for k in range(16):
                        vk = jnp.full((16,), vals[k])
                        e = off + k
                        for j in range(8):
                            slc = (e, pl.ds(j * 16, 16))
                            rbx[slc] = rbx[slc] * vk

            for i in range(4):
                fire_e(i, i)

            @pl.loop(0, NT // K)
            def _(u):
                t0 = u * K
                dg = [None] * K
                ds = [None] * K
                for i in range(K):
                    if i >= 2:
                        ds[i - 2].wait()
                    wait_e(i % 4)
                    dg[i] = fire_g(i % 4, i % 2)
                    if i >= 1:
                        dg[i - 1].wait()
                        scale((i - 1) % 4, (i - 1) % 2)
                        ds[i - 1] = fire_s((i - 1) % 2)
                        fire_e(t0 + i + 3, (i - 1) % 4)
                dg[K - 1].wait()
                scale((K - 1) % 4, (K - 1) % 2)
                ds[K - 1] = fire_s((K - 1) % 2)
                fire_e(t0 + K + 3, (K - 1) % 4)
                ds[K - 2].wait()
                ds[K - 1].wait()

            for b in range(4):
                wait_e(b)

            plsc.subcore_barrier()

            @pl.loop(0, 4)
            def _(t):
                pltpu.sync_copy(acc.at[pl.ds(base + t * CHUNK, CHUNK)],
                                out_hbm.at[pl.ds(base + t * CHUNK, CHUNK)])
            pltpu.sync_copy(acc.at[pl.ds(base + 4 * CHUNK, ROWS_PER_SUB - 4 * CHUNK)],
                            out_hbm.at[pl.ds(base + 4 * CHUNK, ROWS_PER_SUB - 4 * CHUNK)])

            @pl.when(sid == NSUB - 1)
            def _():
                pltpu.sync_copy(acc.at[pl.ds(NSUB * ROWS_PER_SUB, 16)],
                                out_hbm.at[pl.ds(NSUB * ROWS_PER_SUB, 16)])

        @pl.when(cid == 0)
        def _():
            process(ms_hbm, edm_hbm, mo_hbm)

        @pl.when(cid == 1)
        def _():
            process(vs_hbm, edv_hbm, vo_hbm)

    return sc_kernel(m_s, v_s, ed_m, ed_v)


def kernel(mean, var, edge_index, adj0_values, adj1_values,
           W_mean, b_mean, W_var, b_var):
    m_s, v_s = _dense(mean, var, W_mean.T, b_mean[None, :], W_var.T,
                      b_var[None, :])
    row = edge_index[0]
    col = edge_index[1]
    ed_m = _pack_edges(col, row, adj0_values)
    ed_v = _pack_edges(col, row, adj1_values)
    return _sc_spmm(m_s, v_s, ed_m, ed_v)
